# Initial kernel scaffold; baseline (speedup 1.0000x reference)
import jax
import jax.numpy as jnp
from jax.experimental import pallas as pl


def kernel(input_ids, embed_weight):
    raise NotImplementedError("write your pallas kernel here")


# speedup vs baseline: 1.4394x; 1.4394x over previous
import functools

import jax
import jax.numpy as jnp
from jax import lax
from jax.experimental import pallas as pl
from jax.experimental.pallas import tpu as pltpu
from jax.experimental.pallas import tpu_sc as plsc

EMBED_DIM = 2048
NUM_CORES = 2
NUM_SUBCORES = 16
NUM_TILES = NUM_CORES * NUM_SUBCORES
ROWS_PER_CHUNK = 16


@functools.partial(jax.jit, static_argnames=("num_chunks",))
def _sc_embed(embed_weight, idx, num_chunks):
    rows_per_tile = num_chunks * ROWS_PER_CHUNK
    total_rows = NUM_TILES * rows_per_tile
    mesh = plsc.VectorSubcoreMesh(core_axis_name="c", subcore_axis_name="s")

    @functools.partial(
        pl.kernel,
        out_type=jax.ShapeDtypeStruct((total_rows, EMBED_DIM), jnp.float32),
        mesh=mesh,
        scratch_types=[
            pltpu.VMEM((num_chunks, ROWS_PER_CHUNK), jnp.int32),
            pltpu.VMEM((ROWS_PER_CHUNK, EMBED_DIM), jnp.float32),
        ],
    )
    def k(table_hbm, idx_hbm, out_hbm, idx_v, rows_v):
        wid = lax.axis_index("s") * NUM_CORES + lax.axis_index("c")
        pltpu.sync_copy(idx_hbm.at[wid], idx_v)
        base = wid * rows_per_tile

        @pl.loop(0, num_chunks)
        def _(j):
            pltpu.sync_copy(table_hbm.at[idx_v.at[j]], rows_v)
            pltpu.sync_copy(
                rows_v, out_hbm.at[pl.ds(base + j * ROWS_PER_CHUNK, ROWS_PER_CHUNK)]
            )

    return k(embed_weight, idx)


def kernel(input_ids, embed_weight):
    batch, seq = input_ids.shape
    total = batch * seq
    num_chunks = total // (NUM_TILES * ROWS_PER_CHUNK)
    idx = input_ids.reshape(NUM_TILES, num_chunks, ROWS_PER_CHUNK)
    out = _sc_embed(embed_weight, idx, num_chunks)
    return out.reshape(batch, seq, embed_weight.shape[1])


# speedup vs baseline: 1.6759x; 1.1643x over previous
import functools

import jax
import jax.numpy as jnp
from jax import lax
from jax.experimental import pallas as pl
from jax.experimental.pallas import tpu as pltpu
from jax.experimental.pallas import tpu_sc as plsc

EMBED_DIM = 2048
NUM_CORES = 2
NUM_SUBCORES = 16
NUM_TILES = NUM_CORES * NUM_SUBCORES
ROWS_PER_CHUNK = 16


@functools.partial(jax.jit, static_argnames=("num_chunks",))
def _sc_embed(embed_weight, idx, num_chunks):
    rows_per_tile = num_chunks * ROWS_PER_CHUNK
    total_rows = NUM_TILES * rows_per_tile
    mesh = plsc.VectorSubcoreMesh(core_axis_name="c", subcore_axis_name="s")

    @functools.partial(
        pl.kernel,
        out_type=jax.ShapeDtypeStruct((total_rows, EMBED_DIM), jnp.float32),
        mesh=mesh,
        scratch_types=[
            pltpu.VMEM((num_chunks, ROWS_PER_CHUNK), jnp.int32),
            pltpu.VMEM((ROWS_PER_CHUNK, EMBED_DIM), jnp.float32),
            pltpu.VMEM((ROWS_PER_CHUNK, EMBED_DIM), jnp.float32),
            pltpu.SemaphoreType.DMA,
            pltpu.SemaphoreType.DMA,
        ],
    )
    def k(table_hbm, idx_hbm, out_hbm, idx_v, buf0, buf1, sem0, sem1):
        wid = lax.axis_index("s") * NUM_CORES + lax.axis_index("c")
        pltpu.sync_copy(idx_hbm.at[wid], idx_v)
        base = wid * rows_per_tile
        R = ROWS_PER_CHUNK

        def fire(j, buf, sem):
            pltpu.async_copy(table_hbm.at[idx_v.at[j]], buf, sem)

        def drain_and_store(j, buf, sem):
            pltpu.make_async_copy(table_hbm.at[idx_v.at[j]], buf, sem).wait()
            pltpu.sync_copy(buf, out_hbm.at[pl.ds(base + j * R, R)])

        fire(0, buf0, sem0)

        @pl.loop(0, num_chunks - 2, step=2)
        def _(j):
            fire(j + 1, buf1, sem1)
            drain_and_store(j, buf0, sem0)
            fire(j + 2, buf0, sem0)
            drain_and_store(j + 1, buf1, sem1)

        jlast = num_chunks - 2
        fire(jlast + 1, buf1, sem1)
        drain_and_store(jlast, buf0, sem0)
        drain_and_store(jlast + 1, buf1, sem1)

    return k(embed_weight, idx)


def kernel(input_ids, embed_weight):
    batch, seq = input_ids.shape
    total = batch * seq
    num_chunks = total // (NUM_TILES * ROWS_PER_CHUNK)
    idx = input_ids.reshape(NUM_TILES, num_chunks, ROWS_PER_CHUNK)
    out = _sc_embed(embed_weight, idx, num_chunks)
    return out.reshape(batch, seq, embed_weight.shape[1])
